# baseline (speedup 1.0000x reference)
import functools

import jax
import jax.numpy as jnp
from jax import lax
from jax.experimental import pallas as pl
from jax.experimental.pallas import tpu as pltpu
from jax.experimental.pallas import tpu_sc as plsc

_CLS = 1_000_000
_FEAT = 64
_B = 16384
_NC = 2
_NS = 16
_L = 16
_NW = _NC * _NS
_BPW = _B // _NW
_CPS = _B // _NS
_TBL = 1 << 20
_TPS = _TBL // _NS
_ZCH = 4096
_CHK = 256


def _make_sc_kernel():
    mesh = plsc.VectorSubcoreMesh(core_axis_name="c", subcore_axis_name="s")

    @functools.partial(
        pl.kernel,
        mesh=mesh,
        out_type=jax.ShapeDtypeStruct((_NW, _L), jnp.float32),
        compiler_params=pltpu.CompilerParams(use_tc_tiling_on_sc=False),
        scratch_types=[
            pltpu.VMEM_SHARED((_TBL,), jnp.float32),
            pltpu.VMEM((_CPS,), jnp.float32),
            pltpu.VMEM((8, 128), jnp.int32),
            pltpu.VMEM((128,), jnp.float32),
            pltpu.VMEM((_ZCH,), jnp.float32),
            pltpu.VMEM((_BPW,), jnp.float32),
            pltpu.VMEM((_BPW,), jnp.float32),
            pltpu.VMEM((_CHK, _FEAT), jnp.float32),
            pltpu.VMEM((_CHK, _FEAT), jnp.float32),
            pltpu.VMEM((_L,), jnp.float32),
            pltpu.SemaphoreType.DMA,
        ],
    )
    def center_loss_sc(xs_h, ys_h, ct_h, out_h,
                       table, yf, cidx, ones, zbuf, cnt, wbuf, xsv, rows,
                       outb, sem):
        c = lax.axis_index("c")
        s = lax.axis_index("s")
        wid = s * _NC + c

        z16 = jnp.zeros((_L,), jnp.float32)
        o16 = jnp.ones((_L,), jnp.float32)

        def zfill(i, carry):
            zbuf[pl.ds(i * _L, _L)] = z16
            return carry

        lax.fori_loop(0, _ZCH // _L, zfill, 0)

        def ofill(i, carry):
            ones[pl.ds(i * _L, _L)] = o16
            return carry

        lax.fori_loop(0, 128 // _L, ofill, 0)

        tb = s * _TPS
        for j in range(_TPS // _ZCH):
            pltpu.sync_copy(zbuf, table.at[pl.ds(tb + j * _ZCH, _ZCH)])

        pltpu.sync_copy(ys_h.at[pl.ds(s * _CPS, _CPS)], yf)
        for j in range(8):
            def conv(l, carry, j=j):
                v = yf[pl.ds(j * 128 + l * _L, _L)]
                cidx[j, pl.ds(l * _L, _L)] = v.astype(jnp.int32)
                return carry

            lax.fori_loop(0, 128 // _L, conv, 0)

        plsc.subcore_barrier()

        for j in range(8):
            pltpu.sync_copy(ones, table.at[cidx.at[j]], add=True)

        plsc.subcore_barrier()

        base = s * _CPS + c * _BPW
        r0 = c * 4
        for j in range(4):
            pltpu.sync_copy(table.at[cidx.at[r0 + j]],
                            cnt.at[pl.ds(j * 128, 128)])

        for g in range(_BPW // _L):
            wbuf[pl.ds(g * _L, _L)] = 0.5 / (cnt[pl.ds(g * _L, _L)] + 1.0)

        lacc = jnp.zeros((_L,), jnp.float32)
        for half in range(_BPW // _CHK):
            copies = []
            for j in range(_CHK // 128):
                row = cidx.at[r0 + half * (_CHK // 128) + j]
                copies.append(
                    pltpu.async_copy(ct_h.at[row],
                                     rows.at[pl.ds(j * 128, 128)], sem))
            pltpu.sync_copy(xs_h.at[pl.ds(base + half * _CHK, _CHK)], xsv)
            for cp in copies:
                cp.wait()

            def group(g, a, half=half):
                wv = wbuf[pl.ds(half * _CHK + g * _L, _L)]
                for l in range(_L):
                    w = jnp.full((_L,), wv[l], jnp.float32)
                    e = g * _L + l
                    for v in range(_FEAT // _L):
                        d = (xsv[e, pl.ds(v * _L, _L)]
                             - rows[e, pl.ds(v * _L, _L)])
                        a = a + w * (d * d)
                return a

            lacc = lax.fori_loop(0, _CHK // _L, group, lacc)

        outb[...] = lacc
        pltpu.sync_copy(outb, out_h.at[wid])

    return center_loss_sc


_center_loss = _make_sc_kernel()


def kernel(xs, ys, center):
    partials = _center_loss(xs, ys, center)
    return jnp.sum(partials)


# speedup vs baseline: 1.0061x; 1.0061x over previous
import functools

import jax
import jax.numpy as jnp
from jax import lax
from jax.experimental import pallas as pl
from jax.experimental.pallas import tpu as pltpu
from jax.experimental.pallas import tpu_sc as plsc

_CLS = 1_000_000
_FEAT = 64
_B = 16384
_NC = 2
_NS = 16
_L = 16
_NW = _NC * _NS
_BPW = _B // _NW
_CPS = _B // _NS
_TBL = 1 << 20
_TPS = _TBL // _NS
_ZCH = 4096
_CHK = 256


def _make_sc_kernel():
    mesh = plsc.VectorSubcoreMesh(core_axis_name="c", subcore_axis_name="s")

    @functools.partial(
        pl.kernel,
        mesh=mesh,
        out_type=jax.ShapeDtypeStruct((_NW, _L), jnp.float32),
        compiler_params=pltpu.CompilerParams(use_tc_tiling_on_sc=False),
        scratch_types=[
            pltpu.VMEM_SHARED((_TBL,), jnp.float32),
            pltpu.VMEM((_CPS,), jnp.float32),
            pltpu.VMEM((8, 128), jnp.int32),
            pltpu.VMEM((128,), jnp.float32),
            pltpu.VMEM((128,), jnp.float32),
            pltpu.VMEM((_BPW,), jnp.float32),
            pltpu.VMEM((_BPW,), jnp.float32),
            pltpu.VMEM((_CHK, _FEAT), jnp.float32),
            pltpu.VMEM((_CHK, _FEAT), jnp.float32),
            pltpu.VMEM((_L,), jnp.float32),
            pltpu.SemaphoreType.DMA,
        ],
    )
    def center_loss_sc(xs_h, ys_h, ct_h, out_h,
                       table, yf, cidx, ones, zbuf, cnt, wbuf, xsv, rows,
                       outb, sem):
        c = lax.axis_index("c")
        s = lax.axis_index("s")
        wid = s * _NC + c

        z16 = jnp.zeros((_L,), jnp.float32)
        o16 = jnp.ones((_L,), jnp.float32)

        def ofill(i, carry):
            ones[pl.ds(i * _L, _L)] = o16
            zbuf[pl.ds(i * _L, _L)] = z16
            return carry

        lax.fori_loop(0, 128 // _L, ofill, 0)

        pltpu.sync_copy(ys_h.at[pl.ds(s * _CPS, _CPS)], yf)
        for j in range(8):
            def conv(l, carry, j=j):
                v = yf[pl.ds(j * 128 + l * _L, _L)]
                cidx[j, pl.ds(l * _L, _L)] = v.astype(jnp.int32)
                return carry

            lax.fori_loop(0, 128 // _L, conv, 0)

        for j in range(8):
            pltpu.sync_copy(zbuf, table.at[cidx.at[j]])
        plsc.subcore_barrier()
        for j in range(8):
            pltpu.sync_copy(ones, table.at[cidx.at[j]], add=True)
        plsc.subcore_barrier()

        base = s * _CPS + c * _BPW
        r0 = c * 4
        for j in range(4):
            pltpu.sync_copy(table.at[cidx.at[r0 + j]],
                            cnt.at[pl.ds(j * 128, 128)])

        for g in range(_BPW // _L):
            wbuf[pl.ds(g * _L, _L)] = 0.5 / (cnt[pl.ds(g * _L, _L)] + 1.0)

        lacc = jnp.zeros((_L,), jnp.float32)
        for half in range(_BPW // _CHK):
            copies = []
            for j in range(_CHK // 128):
                row = cidx.at[r0 + half * (_CHK // 128) + j]
                copies.append(
                    pltpu.async_copy(ct_h.at[row],
                                     rows.at[pl.ds(j * 128, 128)], sem))
            pltpu.sync_copy(xs_h.at[pl.ds(base + half * _CHK, _CHK)], xsv)
            for cp in copies:
                cp.wait()

            def group(g, a, half=half):
                wv = wbuf[pl.ds(half * _CHK + g * _L, _L)]
                for l in range(_L):
                    w = jnp.full((_L,), wv[l], jnp.float32)
                    e = g * _L + l
                    for v in range(_FEAT // _L):
                        d = (xsv[e, pl.ds(v * _L, _L)]
                             - rows[e, pl.ds(v * _L, _L)])
                        a = a + w * (d * d)
                return a

            lacc = lax.fori_loop(0, _CHK // _L, group, lacc)

        outb[...] = lacc
        pltpu.sync_copy(outb, out_h.at[wid])

    return center_loss_sc


_center_loss = _make_sc_kernel()


def kernel(xs, ys, center):
    partials = _center_loss(xs, ys, center)
    return jnp.sum(partials)


# speedup vs baseline: 1.6727x; 1.6626x over previous
import functools

import jax
import jax.numpy as jnp
from jax import lax
from jax.experimental import pallas as pl
from jax.experimental.pallas import tpu as pltpu
from jax.experimental.pallas import tpu_sc as plsc

_CLS = 1_000_000
_FEAT = 64
_B = 16384
_NC = 2
_NS = 16
_L = 16
_NW = _NC * _NS
_BPW = _B // _NW
_CPS = _B // _NS
_TBL = 1 << 20
_CHK = 128


def _make_sc_kernel():
    mesh = plsc.VectorSubcoreMesh(core_axis_name="c", subcore_axis_name="s")

    @functools.partial(
        pl.kernel,
        mesh=mesh,
        out_type=jax.ShapeDtypeStruct((_NW * _L,), jnp.float32),
        scratch_types=[
            pltpu.VMEM_SHARED((_TBL,), jnp.float32),
            pltpu.VMEM((_CPS,), jnp.float32),
            pltpu.VMEM((8, 128), jnp.int32),
            pltpu.VMEM((128,), jnp.float32),
            pltpu.VMEM((128,), jnp.float32),
            pltpu.VMEM((_BPW,), jnp.float32),
            pltpu.VMEM((_BPW,), jnp.float32),
            pltpu.VMEM((_CHK, _FEAT), jnp.float32),
            pltpu.VMEM((_CHK, _FEAT), jnp.float32),
            pltpu.VMEM((_L,), jnp.float32),
            pltpu.SemaphoreType.DMA,
        ],
    )
    def center_loss_sc(xs_h, ys_h, ct_h, out_h,
                       table, yf, cidx, ones, zbuf, cnt, wbuf, xsv, rows,
                       outb, sem):
        c = lax.axis_index("c")
        s = lax.axis_index("s")
        wid = s * _NC + c

        z16 = jnp.zeros((_L,), jnp.float32)
        o16 = jnp.ones((_L,), jnp.float32)

        def ofill(i, carry):
            ones[pl.ds(i * _L, _L)] = o16
            zbuf[pl.ds(i * _L, _L)] = z16
            return carry

        lax.fori_loop(0, 128 // _L, ofill, 0)

        pltpu.sync_copy(ys_h.at[pl.ds(s * _CPS, _CPS)], yf)
        for j in range(8):
            def conv(l, carry, j=j):
                v = yf[pl.ds(j * 128 + l * _L, _L)]
                cidx[j, pl.ds(l * _L, _L)] = v.astype(jnp.int32)
                return carry

            lax.fori_loop(0, 128 // _L, conv, 0)

        for j in range(8):
            pltpu.sync_copy(zbuf, table.at[cidx.at[j]])
        plsc.subcore_barrier()
        for j in range(8):
            pltpu.sync_copy(ones, table.at[cidx.at[j]], add=True)
        plsc.subcore_barrier()

        base = s * _CPS + c * _BPW
        r0 = c * 4
        for j in range(4):
            pltpu.sync_copy(table.at[cidx.at[r0 + j]],
                            cnt.at[pl.ds(j * 128, 128)])
        for g in range(_BPW // _L):
            wbuf[pl.ds(g * _L, _L)] = 0.5 / (cnt[pl.ds(g * _L, _L)] + 1.0)

        lacc = jnp.zeros((_L,), jnp.float32)
        for k in range(_BPW // _CHK):
            def fire(g, carry, k=k):
                idv = cidx[r0 + k, pl.ds(g * _L, _L)]
                for l in range(_L):
                    pltpu.async_copy(ct_h.at[idv[l]],
                                     rows.at[g * _L + l], sem)
                return carry

            lax.fori_loop(0, _CHK // _L, fire, 0)
            pltpu.sync_copy(xs_h.at[pl.ds(base + k * _CHK, _CHK)], xsv)
            pltpu.make_async_copy(ct_h.at[pl.ds(0, _CHK)], rows, sem).wait()

            def group(g, a, k=k):
                wv = wbuf[pl.ds(k * _CHK + g * _L, _L)]
                for l in range(_L):
                    w = jnp.full((_L,), wv[l], jnp.float32)
                    e = g * _L + l
                    for v in range(_FEAT // _L):
                        d = (xsv[e, pl.ds(v * _L, _L)]
                             - rows[e, pl.ds(v * _L, _L)])
                        a = a + w * (d * d)
                return a

            lacc = lax.fori_loop(0, _CHK // _L, group, lacc)

        outb[...] = lacc
        pltpu.sync_copy(outb, out_h.at[pl.ds(wid * _L, _L)])

    return center_loss_sc


_center_loss = _make_sc_kernel()


def kernel(xs, ys, center):
    partials = _center_loss(xs, ys, center)
    return jnp.sum(partials)


# speedup vs baseline: 1.6844x; 1.0070x over previous
import functools

import jax
import jax.numpy as jnp
from jax import lax
from jax.experimental import pallas as pl
from jax.experimental.pallas import tpu as pltpu
from jax.experimental.pallas import tpu_sc as plsc

_CLS = 1_000_000
_FEAT = 64
_PITCH = 128
_B = 16384
_NC = 2
_NS = 16
_L = 16
_NW = _NC * _NS
_BPW = _B // _NW
_CPS = _B // _NS
_TBL = 1 << 20
_CHK = 128
_NCHK = _BPW // _CHK


def _make_sc_kernel():
    mesh = plsc.VectorSubcoreMesh(core_axis_name="c", subcore_axis_name="s")

    @functools.partial(
        pl.kernel,
        mesh=mesh,
        out_type=jax.ShapeDtypeStruct((_NW * _L,), jnp.float32),
        scratch_types=[
            pltpu.VMEM_SHARED((_TBL,), jnp.float32),
            pltpu.VMEM((_CPS,), jnp.float32),
            pltpu.VMEM((8, 128), jnp.int32),
            pltpu.VMEM((128,), jnp.float32),
            pltpu.VMEM((128,), jnp.float32),
            pltpu.VMEM((_BPW,), jnp.float32),
            pltpu.VMEM((_BPW,), jnp.float32),
            pltpu.VMEM((_CHK, _FEAT), jnp.float32),
            pltpu.VMEM((_CHK, _FEAT), jnp.float32),
            pltpu.VMEM((_L,), jnp.float32),
            pltpu.SemaphoreType.DMA,
            pltpu.SemaphoreType.DMA,
            pltpu.SemaphoreType.DMA,
            pltpu.SemaphoreType.DMA,
        ],
    )
    def center_loss_sc(xs_h, ys_h, ct_h, out_h,
                       table, yf, cidx, ones, zbuf, cnt, wbuf,
                       xsv, rows, outb, sem0, sem1, sem2, sem3):
        sems = (sem0, sem1, sem2, sem3)
        c = lax.axis_index("c")
        s = lax.axis_index("s")
        wid = s * _NC + c

        z16 = jnp.zeros((_L,), jnp.float32)
        o16 = jnp.ones((_L,), jnp.float32)

        def ofill(i, carry):
            ones[pl.ds(i * _L, _L)] = o16
            zbuf[pl.ds(i * _L, _L)] = z16
            return carry

        lax.fori_loop(0, 128 // _L, ofill, 0)

        pltpu.sync_copy(ys_h.at[pl.ds(s * _CPS, _CPS)], yf)
        for j in range(8):
            def conv(l, carry, j=j):
                v = yf[pl.ds(j * 128 + l * _L, _L)]
                cidx[j, pl.ds(l * _L, _L)] = v.astype(jnp.int32)
                return carry

            lax.fori_loop(0, 128 // _L, conv, 0)

        for j in range(8):
            pltpu.sync_copy(zbuf, table.at[cidx.at[j]])
        plsc.subcore_barrier()
        for j in range(8):
            pltpu.sync_copy(ones, table.at[cidx.at[j]], add=True)
        plsc.subcore_barrier()

        base = s * _CPS + c * _BPW
        r0 = c * 4
        for j in range(4):
            pltpu.sync_copy(table.at[cidx.at[r0 + j]],
                            cnt.at[pl.ds(j * 128, 128)])
        for g in range(_BPW // _L):
            wbuf[pl.ds(g * _L, _L)] = 0.5 / (cnt[pl.ds(g * _L, _L)] + 1.0)

        lacc = jnp.zeros((_L,), jnp.float32)

        def chunk(k, a):
            def fire(g, carry):
                idv = cidx[r0 + k, pl.ds(g * _L, _L)]
                for l in range(_L):
                    pltpu.async_copy(ct_h.at[idv[l]],
                                     rows.at[g * _L + l], sems[l % 4])
                return carry

            lax.fori_loop(0, _CHK // _L, fire, 0)
            pltpu.sync_copy(xs_h.at[pl.ds(base + k * _CHK, _CHK)], xsv)
            for q in range(4):
                pltpu.make_async_copy(ct_h.at[pl.ds(0, _CHK // 4)],
                                      rows.at[pl.ds(0, _CHK // 4)],
                                      sems[q]).wait()

            def group(g, aa):
                wv = wbuf[pl.ds(k * _CHK + g * _L, _L)]
                for l in range(_L):
                    w = jnp.full((_L,), wv[l], jnp.float32)
                    e = g * _L + l
                    for v in range(_FEAT // _L):
                        d = (xsv[e, pl.ds(v * _L, _L)]
                             - rows[e, pl.ds(v * _L, _L)])
                        aa = aa + w * (d * d)
                return aa

            return lax.fori_loop(0, _CHK // _L, group, a)

        lacc = lax.fori_loop(0, _NCHK, chunk, lacc)

        outb[...] = lacc
        pltpu.sync_copy(outb, out_h.at[pl.ds(wid * _L, _L)])

    return center_loss_sc


_center_loss = _make_sc_kernel()


def kernel(xs, ys, center):
    partials = _center_loss(xs, ys, center)
    return jnp.sum(partials)
